# baseline (speedup 1.0000x reference)
import dataclasses
import functools

import jax
import jax.numpy as jnp
from jax import lax
from jax.experimental import pallas as pl
from jax.experimental.pallas import tpu as pltpu
from jax.experimental.pallas import tpu_sc as plsc

N = 10000
E = 320000
NC = 2
NS = 16
NW = NC * NS
L = 16

EPW = E // NW
EB = 80
NBLK = EPW // EB
CHUNK = 2000
NP = 10240
RPS = NP // NS
ZR = 128

_mesh = plsc.VectorSubcoreMesh(
    core_axis_name="c", subcore_axis_name="s", num_cores=NC, num_subcores=NS
)

_sc_params = pltpu.CompilerParams()
for _f, _v in (("needs_layout_passes", False), ("use_tc_tiling_on_sc", False)):
    if _f in pltpu.CompilerParams.__dataclass_fields__:
        _sc_params = dataclasses.replace(_sc_params, **{_f: _v})

@functools.partial(
    pl.kernel,
    out_type=jax.ShapeDtypeStruct((NW * N,), jnp.float32),
    mesh=_mesh,
    scratch_types=[
        pltpu.VMEM((N,), jnp.float32),
        pltpu.VMEM((CHUNK,), jnp.int32),
        pltpu.VMEM((CHUNK,), jnp.float32),
    ],
    compiler_params=_sc_params,
)
def _sc_degree(dst_hbm, w_hbm, out_hbm, deg_v, dst_v, w_v):
    cid = lax.axis_index("c")
    sid = lax.axis_index("s")
    wid = cid * NS + sid
    zero16 = jnp.zeros((L,), jnp.float32)

    @pl.loop(0, N, step=L)
    def _(i):
        deg_v[pl.ds(i, L)] = zero16

    base = wid * EPW

    @pl.loop(0, EPW, step=CHUNK)
    def _(off):
        pltpu.sync_copy(dst_hbm.at[pl.ds(base + off, CHUNK)], dst_v)
        pltpu.sync_copy(w_hbm.at[pl.ds(base + off, CHUNK)], w_v)

        @pl.loop(0, CHUNK, step=L)
        def _(j):
            idx = dst_v[pl.ds(j, L)]
            val = w_v[pl.ds(j, L)]
            plsc.addupdate_scatter(deg_v, [idx], val)

    pltpu.sync_copy(deg_v, out_hbm.at[pl.ds(wid * N, N)])


def _make_sc_aggregate(D):
    nch = D // L

    @functools.partial(
        pl.kernel,
        out_type=jax.ShapeDtypeStruct((NC, NP, D), jnp.float32),
        mesh=_mesh,
        scratch_types=[
            pltpu.VMEM_SHARED((NP, D), jnp.float32),
            pltpu.VMEM((4, 3, EB), jnp.int32),
            pltpu.VMEM((4, EB), jnp.int32),
            pltpu.VMEM((4, EB), jnp.float32),
            pltpu.VMEM((4, EB, D), jnp.float32),
        ] + [pltpu.SemaphoreType.DMA] * 12,
        compiler_params=_sc_params,
    )
    def agg(h_hbm, pk_hbm, out_hbm,
            acc_sh, pk_v, dst_v, w_v, rows_v,
            sg0, sg1, sg2, sg3, ss0, ss1, ss2, ss3, sp0, sp1, sp2, sp3):
        cid = lax.axis_index("c")
        sid = lax.axis_index("s")
        wid = cid * NS + sid
        sg = (sg0, sg1, sg2, sg3)
        ss = (ss0, ss1, ss2, ss3)
        sp = (sp0, sp1, sp2, sp3)

        zero16 = jnp.zeros((L,), jnp.float32)

        @pl.loop(0, EB)
        def _(r):
            for c in range(nch):
                rows_v[0, r, pl.ds(c * L, L)] = zero16

        row0 = sid * RPS

        @pl.loop(0, RPS, step=EB)
        def _(r):
            pltpu.sync_copy(rows_v.at[0], acc_sh.at[pl.ds(row0 + r, EB)])

        plsc.subcore_barrier()

        def issue_pack(buf, blk):
            pltpu.async_copy(pk_hbm.at[wid].at[blk], pk_v.at[buf], sp[buf])

        def wait_pack(buf, blk):
            pltpu.make_async_copy(
                pk_hbm.at[wid].at[blk], pk_v.at[buf], sp[buf]
            ).wait()

        def issue_gather(buf, blk):
            pltpu.async_copy(
                h_hbm.at[pk_v.at[buf, 0]], rows_v.at[buf], sg[buf]
            )

        def wait_gather(buf, blk):
            pltpu.make_async_copy(
                h_hbm.at[pk_v.at[buf, 0]], rows_v.at[buf], sg[buf]
            ).wait()

        def drain_scatter(buf):
            pltpu.make_async_copy(
                rows_v.at[buf], acc_sh.at[dst_v.at[buf]], ss[buf]
            ).wait()

        def stage(buf, blk, first=False):
            if not first:
                drain_scatter(buf)

            @pl.when(blk < NBLK)
            def _():
                wait_pack(buf, blk)
                for cc in range(EB // L):
                    sl = pl.ds(cc * L, L)
                    dst_v[buf, sl] = pk_v[buf, 1, sl]
                    w_v[buf, sl] = plsc.bitcast(pk_v[buf, 2, sl], jnp.float32)
                issue_gather(buf, blk)

        def work(buf, blk):
            wait_gather(buf, blk)

            @pl.when(blk + 4 < NBLK)
            def _():
                issue_pack(buf, blk + 4)

            buf16 = jnp.full((L,), buf, jnp.int32)

            @plsc.parallel_loop(0, EB, unroll=8)
            def _(j):
                wj = plsc.load_gather(
                    w_v, [buf16, jnp.full((L,), 0, jnp.int32) + j]
                )
                for c in range(nch):
                    sl = pl.ds(c * L, L)
                    rows_v[buf, j, sl] = rows_v[buf, j, sl] * wj

            pltpu.async_copy(
                rows_v.at[buf], acc_sh.at[dst_v.at[buf]], ss[buf], add=True
            )

        for b in range(4):
            issue_pack(b, b)
        for b in range(4):
            stage(b, b, first=True)

        @pl.loop(0, NBLK - 4, step=4)
        def _(k):
            work(0, k)
            work(1, k + 1)
            stage(0, k + 4)
            work(2, k + 2)
            stage(1, k + 5)
            work(3, k + 3)
            stage(2, k + 6)
            stage(3, k + 7)

        work(0, NBLK - 1)
        drain_scatter(0)

        plsc.subcore_barrier()

        pltpu.sync_copy(
            acc_sh.at[pl.ds(row0, RPS)],
            out_hbm.at[cid].at[pl.ds(row0, RPS)],
        )

    return agg


_sc_aggregate_128 = _make_sc_aggregate(128)
_sc_aggregate_64 = _make_sc_aggregate(64)


BN = 2000


def _m1_body(x_ref, w_ref, dinv_ref, h_ref):
    h = jnp.dot(x_ref[...], w_ref[...], preferred_element_type=jnp.float32)
    h_ref[...] = h * dinv_ref[...]


def _tc_matmul_scale(x, W, dinv):
    d_out = W.shape[1]
    return pl.pallas_call(
        _m1_body,
        grid=(N // BN,),
        in_specs=[
            pl.BlockSpec((BN, x.shape[1]), lambda i: (i, 0)),
            pl.BlockSpec(W.shape, lambda i: (0, 0)),
            pl.BlockSpec((BN, 1), lambda i: (i, 0)),
        ],
        out_specs=pl.BlockSpec((BN, d_out), lambda i: (i, 0)),
        out_shape=jax.ShapeDtypeStruct((N, d_out), jnp.float32),
    )(x, W, dinv)


def _m2_body(p0_ref, p1_ref, h1_ref, dinv_ref, b1_ref, w2_ref, h2_ref):
    dinv = dinv_ref[...]
    z = dinv * (p0_ref[...] + p1_ref[...] + h1_ref[...]) + b1_ref[...]
    z = jnp.maximum(z, 0.0)
    h2 = jnp.dot(z, w2_ref[...], preferred_element_type=jnp.float32)
    h2_ref[...] = h2 * dinv


def _tc_layer2(p0, p1, h1, dinv, b1, W2):
    d_in, d_out = W2.shape
    return pl.pallas_call(
        _m2_body,
        grid=(N // BN,),
        in_specs=[
            pl.BlockSpec((BN, d_in), lambda i: (i, 0)),
            pl.BlockSpec((BN, d_in), lambda i: (i, 0)),
            pl.BlockSpec((BN, d_in), lambda i: (i, 0)),
            pl.BlockSpec((BN, 1), lambda i: (i, 0)),
            pl.BlockSpec((1, d_in), lambda i: (0, 0)),
            pl.BlockSpec(W2.shape, lambda i: (0, 0)),
        ],
        out_specs=pl.BlockSpec((BN, d_out), lambda i: (i, 0)),
        out_shape=jax.ShapeDtypeStruct((N, d_out), jnp.float32),
    )(p0, p1, h1, dinv, b1, W2)


def _fin_body(q0_ref, q1_ref, h2_ref, dinv_ref, b2_ref, out_ref):
    out_ref[...] = (
        dinv_ref[...] * (q0_ref[...] + q1_ref[...] + h2_ref[...]) + b2_ref[...]
    )


def _tc_finish(q0, q1, h2, dinv, b2):
    d_out = h2.shape[1]
    return pl.pallas_call(
        _fin_body,
        grid=(N // BN,),
        in_specs=[
            pl.BlockSpec((BN, d_out), lambda i: (i, 0)),
            pl.BlockSpec((BN, d_out), lambda i: (i, 0)),
            pl.BlockSpec((BN, d_out), lambda i: (i, 0)),
            pl.BlockSpec((BN, 1), lambda i: (i, 0)),
            pl.BlockSpec((1, d_out), lambda i: (0, 0)),
        ],
        out_specs=pl.BlockSpec((BN, d_out), lambda i: (i, 0)),
        out_shape=jax.ShapeDtypeStruct((N, d_out), jnp.float32),
    )(q0, q1, h2, dinv, b2)


def kernel(x, edge_index, edge_weight, W1, b1, W2, b2):
    src = edge_index[0]
    dst = edge_index[1]

    degp = _sc_degree(dst, edge_weight)
    deg = 1.0 + jnp.sum(degp.reshape(NW, N), axis=0)
    dinv = lax.rsqrt(deg).reshape(N, 1)

    pk = jnp.stack(
        [
            src.reshape(NW, NBLK, EB),
            dst.reshape(NW, NBLK, EB),
            lax.bitcast_convert_type(edge_weight, jnp.int32).reshape(NW, NBLK, EB),
        ],
        axis=2,
    )

    h1 = _tc_matmul_scale(x, W1, dinv)
    p = _sc_aggregate_128(h1, pk)
    h2 = _tc_layer2(p[0, :N], p[1, :N], h1, dinv, b1.reshape(1, -1), W2)
    q = _sc_aggregate_64(h2, pk)
    return _tc_finish(q[0, :N], q[1, :N], h2, dinv, b2.reshape(1, -1))


# speedup vs baseline: 1.0020x; 1.0020x over previous
import dataclasses
import functools

import jax
import jax.numpy as jnp
from jax import lax
from jax.experimental import pallas as pl
from jax.experimental.pallas import tpu as pltpu
from jax.experimental.pallas import tpu_sc as plsc

N = 10000
E = 320000
NC = 2
NS = 16
NW = NC * NS
L = 16

EPW = E // NW
EB = 80
NBLK = EPW // EB
CHUNK = 2000
NP = 10240
RPS = NP // NS
ZR = 128

_mesh = plsc.VectorSubcoreMesh(
    core_axis_name="c", subcore_axis_name="s", num_cores=NC, num_subcores=NS
)

_sc_params = pltpu.CompilerParams()
for _f, _v in (("needs_layout_passes", False), ("use_tc_tiling_on_sc", False)):
    if _f in pltpu.CompilerParams.__dataclass_fields__:
        _sc_params = dataclasses.replace(_sc_params, **{_f: _v})

@functools.partial(
    pl.kernel,
    out_type=jax.ShapeDtypeStruct((NW * N,), jnp.float32),
    mesh=_mesh,
    scratch_types=[
        pltpu.VMEM((N,), jnp.float32),
        pltpu.VMEM((CHUNK,), jnp.int32),
        pltpu.VMEM((CHUNK,), jnp.float32),
    ],
    compiler_params=_sc_params,
)
def _sc_degree(dst_hbm, w_hbm, out_hbm, deg_v, dst_v, w_v):
    cid = lax.axis_index("c")
    sid = lax.axis_index("s")
    wid = cid * NS + sid
    zero16 = jnp.zeros((L,), jnp.float32)

    @pl.loop(0, N, step=L)
    def _(i):
        deg_v[pl.ds(i, L)] = zero16

    base = wid * EPW

    @pl.loop(0, EPW, step=CHUNK)
    def _(off):
        pltpu.sync_copy(dst_hbm.at[pl.ds(base + off, CHUNK)], dst_v)
        pltpu.sync_copy(w_hbm.at[pl.ds(base + off, CHUNK)], w_v)

        @pl.loop(0, CHUNK, step=L)
        def _(j):
            idx = dst_v[pl.ds(j, L)]
            val = w_v[pl.ds(j, L)]
            plsc.addupdate_scatter(deg_v, [idx], val)

    pltpu.sync_copy(deg_v, out_hbm.at[pl.ds(wid * N, N)])


def _make_sc_aggregate(D):
    nch = D // L

    @functools.partial(
        pl.kernel,
        out_type=jax.ShapeDtypeStruct((NC, NP, D), jnp.float32),
        mesh=_mesh,
        scratch_types=[
            pltpu.VMEM_SHARED((NP, D), jnp.float32),
            pltpu.VMEM((4, 3, EB), jnp.int32),
            pltpu.VMEM((4, EB), jnp.int32),
            pltpu.VMEM((4, EB), jnp.float32),
            pltpu.VMEM((4, EB, D), jnp.float32),
        ] + [pltpu.SemaphoreType.DMA] * 12,
        compiler_params=_sc_params,
    )
    def agg(h_hbm, pk_hbm, out_hbm,
            acc_sh, pk_v, dst_v, w_v, rows_v,
            sg0, sg1, sg2, sg3, ss0, ss1, ss2, ss3, sp0, sp1, sp2, sp3):
        cid = lax.axis_index("c")
        sid = lax.axis_index("s")
        wid = cid * NS + sid
        sg = (sg0, sg1, sg2, sg3)
        ss = (ss0, ss1, ss2, ss3)
        sp = (sp0, sp1, sp2, sp3)

        zero16 = jnp.zeros((L,), jnp.float32)

        @pl.loop(0, EB)
        def _(r):
            for c in range(nch):
                rows_v[0, r, pl.ds(c * L, L)] = zero16

        row0 = sid * RPS

        @pl.loop(0, RPS, step=EB)
        def _(r):
            pltpu.sync_copy(rows_v.at[0], acc_sh.at[pl.ds(row0 + r, EB)])

        plsc.subcore_barrier()

        def issue_pack(buf, blk):
            pltpu.async_copy(pk_hbm.at[wid].at[blk], pk_v.at[buf], sp[buf])

        def wait_pack(buf, blk):
            pltpu.make_async_copy(
                pk_hbm.at[wid].at[blk], pk_v.at[buf], sp[buf]
            ).wait()

        def issue_gather(buf, blk):
            pltpu.async_copy(
                h_hbm.at[pk_v.at[buf, 0]], rows_v.at[buf], sg[buf]
            )

        def wait_gather(buf, blk):
            pltpu.make_async_copy(
                h_hbm.at[pk_v.at[buf, 0]], rows_v.at[buf], sg[buf]
            ).wait()

        def drain_scatter(buf):
            pltpu.make_async_copy(
                rows_v.at[buf], acc_sh.at[dst_v.at[buf]], ss[buf]
            ).wait()

        def stage(buf, blk, first=False):
            if not first:
                drain_scatter(buf)

            @pl.when(blk < NBLK)
            def _():
                wait_pack(buf, blk)
                for cc in range(EB // L):
                    sl = pl.ds(cc * L, L)
                    dst_v[buf, sl] = pk_v[buf, 1, sl]
                    w_v[buf, sl] = plsc.bitcast(pk_v[buf, 2, sl], jnp.float32)
                issue_gather(buf, blk)

        def work(buf, blk):
            wait_gather(buf, blk)

            @pl.when(blk + 4 < NBLK)
            def _():
                issue_pack(buf, blk + 4)

            buf16 = jnp.full((L,), buf, jnp.int32)

            @plsc.parallel_loop(0, EB, unroll=4)
            def _(j):
                wj = plsc.load_gather(
                    w_v, [buf16, jnp.full((L,), 0, jnp.int32) + j]
                )
                for c in range(nch):
                    sl = pl.ds(c * L, L)
                    rows_v[buf, j, sl] = rows_v[buf, j, sl] * wj

            pltpu.async_copy(
                rows_v.at[buf], acc_sh.at[dst_v.at[buf]], ss[buf], add=True
            )

        for b in range(4):
            issue_pack(b, b)
        for b in range(4):
            stage(b, b, first=True)

        @pl.loop(0, NBLK - 4, step=4)
        def _(k):
            work(0, k)
            work(1, k + 1)
            stage(0, k + 4)
            work(2, k + 2)
            stage(1, k + 5)
            work(3, k + 3)
            stage(2, k + 6)
            stage(3, k + 7)

        work(0, NBLK - 1)
        drain_scatter(0)

        plsc.subcore_barrier()

        pltpu.sync_copy(
            acc_sh.at[pl.ds(row0, RPS)],
            out_hbm.at[cid].at[pl.ds(row0, RPS)],
        )

    return agg


_sc_aggregate_128 = _make_sc_aggregate(128)
_sc_aggregate_64 = _make_sc_aggregate(64)


BN = 2000


def _m1_body(x_ref, w_ref, dinv_ref, h_ref):
    h = jnp.dot(x_ref[...], w_ref[...], preferred_element_type=jnp.float32)
    h_ref[...] = h * dinv_ref[...]


def _tc_matmul_scale(x, W, dinv):
    d_out = W.shape[1]
    return pl.pallas_call(
        _m1_body,
        grid=(N // BN,),
        in_specs=[
            pl.BlockSpec((BN, x.shape[1]), lambda i: (i, 0)),
            pl.BlockSpec(W.shape, lambda i: (0, 0)),
            pl.BlockSpec((BN, 1), lambda i: (i, 0)),
        ],
        out_specs=pl.BlockSpec((BN, d_out), lambda i: (i, 0)),
        out_shape=jax.ShapeDtypeStruct((N, d_out), jnp.float32),
    )(x, W, dinv)


def _m2_body(p0_ref, p1_ref, h1_ref, dinv_ref, b1_ref, w2_ref, h2_ref):
    dinv = dinv_ref[...]
    z = dinv * (p0_ref[...] + p1_ref[...] + h1_ref[...]) + b1_ref[...]
    z = jnp.maximum(z, 0.0)
    h2 = jnp.dot(z, w2_ref[...], preferred_element_type=jnp.float32)
    h2_ref[...] = h2 * dinv


def _tc_layer2(p0, p1, h1, dinv, b1, W2):
    d_in, d_out = W2.shape
    return pl.pallas_call(
        _m2_body,
        grid=(N // BN,),
        in_specs=[
            pl.BlockSpec((BN, d_in), lambda i: (i, 0)),
            pl.BlockSpec((BN, d_in), lambda i: (i, 0)),
            pl.BlockSpec((BN, d_in), lambda i: (i, 0)),
            pl.BlockSpec((BN, 1), lambda i: (i, 0)),
            pl.BlockSpec((1, d_in), lambda i: (0, 0)),
            pl.BlockSpec(W2.shape, lambda i: (0, 0)),
        ],
        out_specs=pl.BlockSpec((BN, d_out), lambda i: (i, 0)),
        out_shape=jax.ShapeDtypeStruct((N, d_out), jnp.float32),
    )(p0, p1, h1, dinv, b1, W2)


def _fin_body(q0_ref, q1_ref, h2_ref, dinv_ref, b2_ref, out_ref):
    out_ref[...] = (
        dinv_ref[...] * (q0_ref[...] + q1_ref[...] + h2_ref[...]) + b2_ref[...]
    )


def _tc_finish(q0, q1, h2, dinv, b2):
    d_out = h2.shape[1]
    return pl.pallas_call(
        _fin_body,
        grid=(N // BN,),
        in_specs=[
            pl.BlockSpec((BN, d_out), lambda i: (i, 0)),
            pl.BlockSpec((BN, d_out), lambda i: (i, 0)),
            pl.BlockSpec((BN, d_out), lambda i: (i, 0)),
            pl.BlockSpec((BN, 1), lambda i: (i, 0)),
            pl.BlockSpec((1, d_out), lambda i: (0, 0)),
        ],
        out_specs=pl.BlockSpec((BN, d_out), lambda i: (i, 0)),
        out_shape=jax.ShapeDtypeStruct((N, d_out), jnp.float32),
    )(q0, q1, h2, dinv, b2)


def kernel(x, edge_index, edge_weight, W1, b1, W2, b2):
    src = edge_index[0]
    dst = edge_index[1]

    degp = _sc_degree(dst, edge_weight)
    deg = 1.0 + jnp.sum(degp.reshape(NW, N), axis=0)
    dinv = lax.rsqrt(deg).reshape(N, 1)

    pk = jnp.stack(
        [
            src.reshape(NW, NBLK, EB),
            dst.reshape(NW, NBLK, EB),
            lax.bitcast_convert_type(edge_weight, jnp.int32).reshape(NW, NBLK, EB),
        ],
        axis=2,
    )

    h1 = _tc_matmul_scale(x, W1, dinv)
    p = _sc_aggregate_128(h1, pk)
    h2 = _tc_layer2(p[0, :N], p[1, :N], h1, dinv, b1.reshape(1, -1), W2)
    q = _sc_aggregate_64(h2, pk)
    return _tc_finish(q[0, :N], q[1, :N], h2, dinv, b2.reshape(1, -1))


# speedup vs baseline: 1.0784x; 1.0763x over previous
import dataclasses
import functools

import jax
import jax.numpy as jnp
from jax import lax
from jax.experimental import pallas as pl
from jax.experimental.pallas import tpu as pltpu
from jax.experimental.pallas import tpu_sc as plsc

N = 10000
E = 320000
NC = 2
NS = 16
NW = NC * NS
L = 16

EPW = E // NW
EB = 80
NBLK = EPW // EB
CHUNK = 2000
NP = 10240
RPS = NP // NS
ZR = 128

_mesh = plsc.VectorSubcoreMesh(
    core_axis_name="c", subcore_axis_name="s", num_cores=NC, num_subcores=NS
)

_sc_params = pltpu.CompilerParams()
for _f, _v in (("needs_layout_passes", False), ("use_tc_tiling_on_sc", False)):
    if _f in pltpu.CompilerParams.__dataclass_fields__:
        _sc_params = dataclasses.replace(_sc_params, **{_f: _v})

PCH = 25


@functools.partial(
    pl.kernel,
    out_type=[
        jax.ShapeDtypeStruct((NW, NBLK, 3, EB), jnp.int32),
        jax.ShapeDtypeStruct((NP,), jnp.float32),
    ],
    mesh=_mesh,
    scratch_types=[
        pltpu.VMEM_SHARED((NS * NP,), jnp.float32),
        pltpu.VMEM_SHARED((NP,), jnp.float32),
        pltpu.VMEM((NP,), jnp.float32),
        pltpu.VMEM((NS, RPS), jnp.float32),
        pltpu.VMEM((CHUNK,), jnp.int32),
        pltpu.VMEM((CHUNK,), jnp.int32),
        pltpu.VMEM((CHUNK,), jnp.float32),
        pltpu.VMEM((PCH, 3, EB), jnp.int32),
    ],
    compiler_params=_sc_params,
)
def _sc_norm(src_hbm, dst_hbm, w_hbm, pk_hbm, selfw_hbm,
             degsh, dinvsh, deg_v, red_v, sv, dv, wv, pkb):
    cid = lax.axis_index("c")
    sid = lax.axis_index("s")
    wid = cid * NS + sid
    zero16 = jnp.zeros((L,), jnp.float32)

    @pl.loop(0, NP, step=L)
    def _(i):
        deg_v[pl.ds(i, L)] = zero16

    base1 = sid * (E // NS)

    @pl.loop(0, E // NS, step=CHUNK)
    def _(off):
        pltpu.sync_copy(dst_hbm.at[pl.ds(base1 + off, CHUNK)], dv)
        pltpu.sync_copy(w_hbm.at[pl.ds(base1 + off, CHUNK)], wv)

        @pl.loop(0, CHUNK, step=L)
        def _(j):
            plsc.addupdate_scatter(deg_v, [dv[pl.ds(j, L)]], wv[pl.ds(j, L)])

    pltpu.sync_copy(deg_v, degsh.at[pl.ds(sid * NP, NP)])
    plsc.subcore_barrier()

    row0 = sid * RPS
    for p in range(NS):
        pltpu.sync_copy(degsh.at[pl.ds(p * NP + row0, RPS)], red_v.at[p])

    @pl.loop(0, RPS, step=L)
    def _(i):
        sl = pl.ds(i, L)
        acc = jnp.full((L,), 1.0, jnp.float32)
        for p in range(NS):
            acc = acc + red_v[p, sl]
        xi = plsc.bitcast(acc, jnp.int32)
        yi = jnp.int32(0x5F3759DF) - lax.shift_right_logical(xi, 1)
        y = plsc.bitcast(yi, jnp.float32)
        hx = acc * 0.5
        for _ in range(3):
            y = y * (1.5 - hx * y * y)
        deg_v[sl] = y
        red_v[0, sl] = y * y

    pltpu.sync_copy(deg_v.at[pl.ds(0, RPS)], dinvsh.at[pl.ds(row0, RPS)])

    @pl.when(cid == 0)
    def _():
        pltpu.sync_copy(red_v.at[0], selfw_hbm.at[pl.ds(row0, RPS)])

    plsc.subcore_barrier()
    pltpu.sync_copy(dinvsh, deg_v)

    base3 = wid * EPW

    @pl.loop(0, NBLK, step=PCH)
    def _(cb):
        e0 = base3 + cb * EB
        pltpu.sync_copy(src_hbm.at[pl.ds(e0, CHUNK)], sv)
        pltpu.sync_copy(dst_hbm.at[pl.ds(e0, CHUNK)], dv)
        pltpu.sync_copy(w_hbm.at[pl.ds(e0, CHUNK)], wv)

        @pl.loop(0, PCH)
        def _(jb):
            for v in range(EB // L):
                sl_e = pl.ds(jb * EB + v * L, L)
                s16 = sv[sl_e]
                d16 = dv[sl_e]
                w16 = wv[sl_e]
                nrm = (w16 * plsc.load_gather(deg_v, [s16])
                       * plsc.load_gather(deg_v, [d16]))
                sl_l = pl.ds(v * L, L)
                pkb[jb, 0, sl_l] = s16
                pkb[jb, 1, sl_l] = d16
                pkb[jb, 2, sl_l] = plsc.bitcast(nrm, jnp.int32)

        pltpu.sync_copy(pkb, pk_hbm.at[wid].at[pl.ds(cb, PCH)])


def _make_sc_aggregate(D):
    nch = D // L

    @functools.partial(
        pl.kernel,
        out_type=jax.ShapeDtypeStruct((NC, NP, D), jnp.float32),
        mesh=_mesh,
        scratch_types=[
            pltpu.VMEM_SHARED((NP, D), jnp.float32),
            pltpu.VMEM((4, 3, EB), jnp.int32),
            pltpu.VMEM((4, EB), jnp.int32),
            pltpu.VMEM((4, EB), jnp.float32),
            pltpu.VMEM((4, EB, D), jnp.float32),
        ] + [pltpu.SemaphoreType.DMA] * 12,
        compiler_params=_sc_params,
    )
    def agg(h_hbm, pk_hbm, out_hbm,
            acc_sh, pk_v, dst_v, w_v, rows_v,
            sg0, sg1, sg2, sg3, ss0, ss1, ss2, ss3, sp0, sp1, sp2, sp3):
        cid = lax.axis_index("c")
        sid = lax.axis_index("s")
        wid = cid * NS + sid
        sg = (sg0, sg1, sg2, sg3)
        ss = (ss0, ss1, ss2, ss3)
        sp = (sp0, sp1, sp2, sp3)

        zero16 = jnp.zeros((L,), jnp.float32)

        @pl.loop(0, EB)
        def _(r):
            for c in range(nch):
                rows_v[0, r, pl.ds(c * L, L)] = zero16

        row0 = sid * RPS

        @pl.loop(0, RPS, step=EB)
        def _(r):
            pltpu.sync_copy(rows_v.at[0], acc_sh.at[pl.ds(row0 + r, EB)])

        plsc.subcore_barrier()

        def issue_pack(buf, blk):
            pltpu.async_copy(pk_hbm.at[wid].at[blk], pk_v.at[buf], sp[buf])

        def wait_pack(buf, blk):
            pltpu.make_async_copy(
                pk_hbm.at[wid].at[blk], pk_v.at[buf], sp[buf]
            ).wait()

        def issue_gather(buf, blk):
            pltpu.async_copy(
                h_hbm.at[pk_v.at[buf, 0]], rows_v.at[buf], sg[buf]
            )

        def wait_gather(buf, blk):
            pltpu.make_async_copy(
                h_hbm.at[pk_v.at[buf, 0]], rows_v.at[buf], sg[buf]
            ).wait()

        def drain_scatter(buf):
            pltpu.make_async_copy(
                rows_v.at[buf], acc_sh.at[dst_v.at[buf]], ss[buf]
            ).wait()

        def stage(buf, blk, first=False):
            if not first:
                drain_scatter(buf)

            @pl.when(blk < NBLK)
            def _():
                wait_pack(buf, blk)
                for cc in range(EB // L):
                    sl = pl.ds(cc * L, L)
                    dst_v[buf, sl] = pk_v[buf, 1, sl]
                    w_v[buf, sl] = plsc.bitcast(pk_v[buf, 2, sl], jnp.float32)
                issue_gather(buf, blk)

        def work(buf, blk):
            wait_gather(buf, blk)

            @pl.when(blk + 4 < NBLK)
            def _():
                issue_pack(buf, blk + 4)

            buf16 = jnp.full((L,), buf, jnp.int32)

            @plsc.parallel_loop(0, EB, unroll=4)
            def _(j):
                wj = plsc.load_gather(
                    w_v, [buf16, jnp.full((L,), 0, jnp.int32) + j]
                )
                for c in range(nch):
                    sl = pl.ds(c * L, L)
                    rows_v[buf, j, sl] = rows_v[buf, j, sl] * wj

            pltpu.async_copy(
                rows_v.at[buf], acc_sh.at[dst_v.at[buf]], ss[buf], add=True
            )

        for b in range(4):
            issue_pack(b, b)
        for b in range(4):
            stage(b, b, first=True)

        @pl.loop(0, NBLK - 4, step=4)
        def _(k):
            work(0, k)
            work(1, k + 1)
            stage(0, k + 4)
            work(2, k + 2)
            stage(1, k + 5)
            work(3, k + 3)
            stage(2, k + 6)
            stage(3, k + 7)

        work(0, NBLK - 1)
        drain_scatter(0)

        plsc.subcore_barrier()

        pltpu.sync_copy(
            acc_sh.at[pl.ds(row0, RPS)],
            out_hbm.at[cid].at[pl.ds(row0, RPS)],
        )

    return agg


_sc_aggregate_128 = _make_sc_aggregate(128)
_sc_aggregate_64 = _make_sc_aggregate(64)


BN = 2000


def _m1_body(x_ref, w_ref, h_ref):
    h_ref[...] = jnp.dot(x_ref[...], w_ref[...],
                         preferred_element_type=jnp.float32)


def _tc_matmul(x, W):
    d_out = W.shape[1]
    return pl.pallas_call(
        _m1_body,
        grid=(N // BN,),
        in_specs=[
            pl.BlockSpec((BN, x.shape[1]), lambda i: (i, 0)),
            pl.BlockSpec(W.shape, lambda i: (0, 0)),
        ],
        out_specs=pl.BlockSpec((BN, d_out), lambda i: (i, 0)),
        out_shape=jax.ShapeDtypeStruct((N, d_out), jnp.float32),
    )(x, W)


def _m2_body(p_ref, h1_ref, sw_ref, b1_ref, w2_ref, h2_ref):
    z = p_ref[0] + p_ref[1] + sw_ref[...] * h1_ref[...] + b1_ref[...]
    z = jnp.maximum(z, 0.0)
    h2_ref[...] = jnp.dot(z, w2_ref[...], preferred_element_type=jnp.float32)


def _tc_layer2(p, h1, selfw, b1, W2):
    d_in, d_out = W2.shape
    return pl.pallas_call(
        _m2_body,
        grid=(N // BN,),
        in_specs=[
            pl.BlockSpec((NC, BN, d_in), lambda i: (0, i, 0)),
            pl.BlockSpec((BN, d_in), lambda i: (i, 0)),
            pl.BlockSpec((BN, 1), lambda i: (i, 0)),
            pl.BlockSpec((1, d_in), lambda i: (0, 0)),
            pl.BlockSpec(W2.shape, lambda i: (0, 0)),
        ],
        out_specs=pl.BlockSpec((BN, d_out), lambda i: (i, 0)),
        out_shape=jax.ShapeDtypeStruct((N, d_out), jnp.float32),
    )(p, h1, selfw, b1, W2)


def _fin_body(q_ref, h2_ref, sw_ref, b2_ref, out_ref):
    out_ref[...] = (
        q_ref[0] + q_ref[1] + sw_ref[...] * h2_ref[...] + b2_ref[...]
    )


def _tc_finish(q, h2, selfw, b2):
    d_out = h2.shape[1]
    return pl.pallas_call(
        _fin_body,
        grid=(N // BN,),
        in_specs=[
            pl.BlockSpec((NC, BN, d_out), lambda i: (0, i, 0)),
            pl.BlockSpec((BN, d_out), lambda i: (i, 0)),
            pl.BlockSpec((BN, 1), lambda i: (i, 0)),
            pl.BlockSpec((1, d_out), lambda i: (0, 0)),
        ],
        out_specs=pl.BlockSpec((BN, d_out), lambda i: (i, 0)),
        out_shape=jax.ShapeDtypeStruct((N, d_out), jnp.float32),
    )(q, h2, selfw, b2)


def kernel(x, edge_index, edge_weight, W1, b1, W2, b2):
    src = edge_index[0]
    dst = edge_index[1]

    pk, selfw = _sc_norm(src, dst, edge_weight)
    selfw2 = selfw.reshape(NP, 1)

    h1 = _tc_matmul(x, W1)
    p = _sc_aggregate_128(h1, pk)
    h2 = _tc_layer2(p, h1, selfw2, b1.reshape(1, -1), W2)
    q = _sc_aggregate_64(h2, pk)
    return _tc_finish(q, h2, selfw2, b2.reshape(1, -1))


# speedup vs baseline: 1.2060x; 1.1184x over previous
import dataclasses
import functools

import jax
import jax.numpy as jnp
from jax import lax
from jax.experimental import pallas as pl
from jax.experimental.pallas import tpu as pltpu
from jax.experimental.pallas import tpu_sc as plsc

N = 10000
E = 320000
NC = 2
NS = 16
NW = NC * NS
L = 16

EPW = E // NW
EB = 80
NBLK = EPW // EB
CHUNK = 2000
NP = 10240
RPS = NP // NS
ZR = 128

_mesh = plsc.VectorSubcoreMesh(
    core_axis_name="c", subcore_axis_name="s", num_cores=NC, num_subcores=NS
)

_sc_params = pltpu.CompilerParams()
for _f, _v in (("needs_layout_passes", False), ("use_tc_tiling_on_sc", False)):
    if _f in pltpu.CompilerParams.__dataclass_fields__:
        _sc_params = dataclasses.replace(_sc_params, **{_f: _v})

PCH = 25


@functools.partial(
    pl.kernel,
    out_type=[
        jax.ShapeDtypeStruct((NW, NBLK, 3, EB), jnp.int32),
        jax.ShapeDtypeStruct((NP,), jnp.float32),
    ],
    mesh=_mesh,
    scratch_types=[
        pltpu.VMEM_SHARED((NS * NP,), jnp.float32),
        pltpu.VMEM_SHARED((NP,), jnp.float32),
        pltpu.VMEM((NP,), jnp.float32),
        pltpu.VMEM((NS, RPS), jnp.float32),
        pltpu.VMEM((2, CHUNK), jnp.int32),
        pltpu.VMEM((2, CHUNK), jnp.int32),
        pltpu.VMEM((2, CHUNK), jnp.float32),
        pltpu.VMEM((PCH, 3, EB), jnp.int32),
        pltpu.SemaphoreType.DMA,
        pltpu.SemaphoreType.DMA,
    ],
    compiler_params=_sc_params,
)
def _sc_norm(ei_hbm, w_hbm, pk_hbm, selfw_hbm,
             degsh, dinvsh, deg_v, red_v, sv, dv, wv, pkb, sma, smb):
    cid = lax.axis_index("c")
    sid = lax.axis_index("s")
    wid = cid * NS + sid
    sm = (sma, smb)
    zero16 = jnp.zeros((L,), jnp.float32)

    @pl.loop(0, NP, step=L)
    def _(i):
        deg_v[pl.ds(i, L)] = zero16

    base1 = sid * (E // NS)
    NCH1 = (E // NS) // CHUNK

    def p1_issue(buf, k):
        off = base1 + k * CHUNK
        pltpu.async_copy(ei_hbm.at[pl.ds(E + off, CHUNK)], dv.at[buf], sm[buf])
        pltpu.async_copy(w_hbm.at[pl.ds(off, CHUNK)], wv.at[buf], sm[buf])

    def p1_wait(buf, k):
        off = base1 + k * CHUNK
        pltpu.make_async_copy(
            ei_hbm.at[pl.ds(E + off, CHUNK)], dv.at[buf], sm[buf]).wait()
        pltpu.make_async_copy(
            w_hbm.at[pl.ds(off, CHUNK)], wv.at[buf], sm[buf]).wait()

    def p1_scatter(buf):
        @pl.loop(0, CHUNK, step=L)
        def _(j):
            plsc.addupdate_scatter(
                deg_v, [dv[buf, pl.ds(j, L)]], wv[buf, pl.ds(j, L)])

    p1_issue(0, 0)

    @pl.loop(0, NCH1, step=2)
    def _(k):
        p1_issue(1, k + 1)
        p1_wait(0, k)
        p1_scatter(0)

        @pl.when(k + 2 < NCH1)
        def _():
            p1_issue(0, k + 2)

        p1_wait(1, k + 1)
        p1_scatter(1)

    pltpu.sync_copy(deg_v, degsh.at[pl.ds(sid * NP, NP)])
    plsc.subcore_barrier()

    row0 = sid * RPS
    for p in range(NS):
        pltpu.async_copy(
            degsh.at[pl.ds(p * NP + row0, RPS)], red_v.at[p], sma)
    for p in range(NS):
        pltpu.make_async_copy(
            degsh.at[pl.ds(p * NP + row0, RPS)], red_v.at[p], sma).wait()

    @pl.loop(0, RPS, step=L)
    def _(i):
        sl = pl.ds(i, L)
        acc = jnp.full((L,), 1.0, jnp.float32)
        for p in range(NS):
            acc = acc + red_v[p, sl]
        xi = plsc.bitcast(acc, jnp.int32)
        yi = jnp.int32(0x5F3759DF) - lax.shift_right_logical(xi, 1)
        y = plsc.bitcast(yi, jnp.float32)
        hx = acc * 0.5
        for _ in range(3):
            y = y * (1.5 - hx * y * y)
        deg_v[sl] = y
        red_v[0, sl] = y * y

    pltpu.sync_copy(deg_v.at[pl.ds(0, RPS)], dinvsh.at[pl.ds(row0, RPS)])

    @pl.when(cid == 0)
    def _():
        pltpu.sync_copy(red_v.at[0], selfw_hbm.at[pl.ds(row0, RPS)])

    plsc.subcore_barrier()
    pltpu.sync_copy(dinvsh, deg_v)

    base3 = wid * EPW
    NCH3 = NBLK // PCH

    def p3_issue(buf, c):
        e0 = base3 + c * CHUNK
        pltpu.async_copy(ei_hbm.at[pl.ds(e0, CHUNK)], sv.at[buf], sm[buf])
        pltpu.async_copy(ei_hbm.at[pl.ds(E + e0, CHUNK)], dv.at[buf], sm[buf])
        pltpu.async_copy(w_hbm.at[pl.ds(e0, CHUNK)], wv.at[buf], sm[buf])

    def p3_wait(buf, c):
        e0 = base3 + c * CHUNK
        pltpu.make_async_copy(
            ei_hbm.at[pl.ds(e0, CHUNK)], sv.at[buf], sm[buf]).wait()
        pltpu.make_async_copy(
            ei_hbm.at[pl.ds(E + e0, CHUNK)], dv.at[buf], sm[buf]).wait()
        pltpu.make_async_copy(
            w_hbm.at[pl.ds(e0, CHUNK)], wv.at[buf], sm[buf]).wait()

    p3_issue(0, 0)
    for c in range(NCH3):
        buf = c % 2
        if c + 1 < NCH3:
            p3_issue(1 - buf, c + 1)
        p3_wait(buf, c)

        @pl.loop(0, PCH)
        def _(jb):
            for v in range(EB // L):
                sl_e = pl.ds(jb * EB + v * L, L)
                s16 = sv[buf, sl_e]
                d16 = dv[buf, sl_e]
                w16 = wv[buf, sl_e]
                nrm = (w16 * plsc.load_gather(deg_v, [s16])
                       * plsc.load_gather(deg_v, [d16]))
                sl_l = pl.ds(v * L, L)
                pkb[jb, 0, sl_l] = s16
                pkb[jb, 1, sl_l] = d16
                pkb[jb, 2, sl_l] = plsc.bitcast(nrm, jnp.int32)

        pltpu.sync_copy(pkb, pk_hbm.at[wid].at[pl.ds(c * PCH, PCH)])


def _make_sc_aggregate(D):
    nch = D // L

    @functools.partial(
        pl.kernel,
        out_type=jax.ShapeDtypeStruct((NC, NP, D), jnp.float32),
        mesh=_mesh,
        scratch_types=[
            pltpu.VMEM_SHARED((NP, D), jnp.float32),
            pltpu.VMEM((4, 3, EB), jnp.int32),
            pltpu.VMEM((4, EB), jnp.int32),
            pltpu.VMEM((4, EB), jnp.float32),
            pltpu.VMEM((4, EB, D), jnp.float32),
        ] + [pltpu.SemaphoreType.DMA] * 12,
        compiler_params=_sc_params,
    )
    def agg(h_hbm, pk_hbm, out_hbm,
            acc_sh, pk_v, dst_v, w_v, rows_v,
            sg0, sg1, sg2, sg3, ss0, ss1, ss2, ss3, sp0, sp1, sp2, sp3):
        cid = lax.axis_index("c")
        sid = lax.axis_index("s")
        wid = cid * NS + sid
        sg = (sg0, sg1, sg2, sg3)
        ss = (ss0, ss1, ss2, ss3)
        sp = (sp0, sp1, sp2, sp3)

        zero16 = jnp.zeros((L,), jnp.float32)

        @pl.loop(0, EB)
        def _(r):
            for c in range(nch):
                rows_v[0, r, pl.ds(c * L, L)] = zero16

        row0 = sid * RPS

        @pl.loop(0, RPS, step=EB)
        def _(r):
            pltpu.sync_copy(rows_v.at[0], acc_sh.at[pl.ds(row0 + r, EB)])

        plsc.subcore_barrier()

        def issue_pack(buf, blk):
            pltpu.async_copy(pk_hbm.at[wid].at[blk], pk_v.at[buf], sp[buf])

        def wait_pack(buf, blk):
            pltpu.make_async_copy(
                pk_hbm.at[wid].at[blk], pk_v.at[buf], sp[buf]
            ).wait()

        def issue_gather(buf, blk):
            pltpu.async_copy(
                h_hbm.at[pk_v.at[buf, 0]], rows_v.at[buf], sg[buf]
            )

        def wait_gather(buf, blk):
            pltpu.make_async_copy(
                h_hbm.at[pk_v.at[buf, 0]], rows_v.at[buf], sg[buf]
            ).wait()

        def drain_scatter(buf):
            pltpu.make_async_copy(
                rows_v.at[buf], acc_sh.at[dst_v.at[buf]], ss[buf]
            ).wait()

        def stage(buf, blk, first=False):
            if not first:
                drain_scatter(buf)

            @pl.when(blk < NBLK)
            def _():
                wait_pack(buf, blk)
                for cc in range(EB // L):
                    sl = pl.ds(cc * L, L)
                    dst_v[buf, sl] = pk_v[buf, 1, sl]
                    w_v[buf, sl] = plsc.bitcast(pk_v[buf, 2, sl], jnp.float32)
                issue_gather(buf, blk)

        def work(buf, blk):
            wait_gather(buf, blk)

            @pl.when(blk + 4 < NBLK)
            def _():
                issue_pack(buf, blk + 4)

            buf16 = jnp.full((L,), buf, jnp.int32)

            @plsc.parallel_loop(0, EB, unroll=4)
            def _(j):
                wj = plsc.load_gather(
                    w_v, [buf16, jnp.full((L,), 0, jnp.int32) + j]
                )
                for c in range(nch):
                    sl = pl.ds(c * L, L)
                    rows_v[buf, j, sl] = rows_v[buf, j, sl] * wj

            pltpu.async_copy(
                rows_v.at[buf], acc_sh.at[dst_v.at[buf]], ss[buf], add=True
            )

        for b in range(4):
            issue_pack(b, b)
        for b in range(4):
            stage(b, b, first=True)

        @pl.loop(0, NBLK - 4, step=4)
        def _(k):
            work(0, k)
            work(1, k + 1)
            stage(0, k + 4)
            work(2, k + 2)
            stage(1, k + 5)
            work(3, k + 3)
            stage(2, k + 6)
            stage(3, k + 7)

        work(0, NBLK - 1)
        drain_scatter(0)

        plsc.subcore_barrier()

        pltpu.sync_copy(
            acc_sh.at[pl.ds(row0, RPS)],
            out_hbm.at[cid].at[pl.ds(row0, RPS)],
        )

    return agg


_sc_aggregate_128 = _make_sc_aggregate(128)
_sc_aggregate_64 = _make_sc_aggregate(64)


BN = 2000


def _m1_body(x_ref, w_ref, h_ref):
    h_ref[...] = jnp.dot(x_ref[...], w_ref[...],
                         preferred_element_type=jnp.float32)


def _tc_matmul(x, W):
    d_out = W.shape[1]
    return pl.pallas_call(
        _m1_body,
        grid=(N // BN,),
        in_specs=[
            pl.BlockSpec((BN, x.shape[1]), lambda i: (i, 0)),
            pl.BlockSpec(W.shape, lambda i: (0, 0)),
        ],
        out_specs=pl.BlockSpec((BN, d_out), lambda i: (i, 0)),
        out_shape=jax.ShapeDtypeStruct((N, d_out), jnp.float32),
    )(x, W)


def _m2_body(p_ref, h1_ref, sw_ref, b1_ref, w2_ref, h2_ref):
    z = p_ref[0] + p_ref[1] + sw_ref[...] * h1_ref[...] + b1_ref[...]
    z = jnp.maximum(z, 0.0)
    h2_ref[...] = jnp.dot(z, w2_ref[...], preferred_element_type=jnp.float32)


def _tc_layer2(p, h1, selfw, b1, W2):
    d_in, d_out = W2.shape
    return pl.pallas_call(
        _m2_body,
        grid=(N // BN,),
        in_specs=[
            pl.BlockSpec((NC, BN, d_in), lambda i: (0, i, 0)),
            pl.BlockSpec((BN, d_in), lambda i: (i, 0)),
            pl.BlockSpec((BN, 1), lambda i: (i, 0)),
            pl.BlockSpec((1, d_in), lambda i: (0, 0)),
            pl.BlockSpec(W2.shape, lambda i: (0, 0)),
        ],
        out_specs=pl.BlockSpec((BN, d_out), lambda i: (i, 0)),
        out_shape=jax.ShapeDtypeStruct((N, d_out), jnp.float32),
    )(p, h1, selfw, b1, W2)


def _fin_body(q_ref, h2_ref, sw_ref, b2_ref, out_ref):
    out_ref[...] = (
        q_ref[0] + q_ref[1] + sw_ref[...] * h2_ref[...] + b2_ref[...]
    )


def _tc_finish(q, h2, selfw, b2):
    d_out = h2.shape[1]
    return pl.pallas_call(
        _fin_body,
        grid=(N // BN,),
        in_specs=[
            pl.BlockSpec((NC, BN, d_out), lambda i: (0, i, 0)),
            pl.BlockSpec((BN, d_out), lambda i: (i, 0)),
            pl.BlockSpec((BN, 1), lambda i: (i, 0)),
            pl.BlockSpec((1, d_out), lambda i: (0, 0)),
        ],
        out_specs=pl.BlockSpec((BN, d_out), lambda i: (i, 0)),
        out_shape=jax.ShapeDtypeStruct((N, d_out), jnp.float32),
    )(q, h2, selfw, b2)


def kernel(x, edge_index, edge_weight, W1, b1, W2, b2):
    pk, selfw = _sc_norm(edge_index.reshape(2 * E), edge_weight)
    selfw2 = selfw.reshape(NP, 1)

    h1 = _tc_matmul(x, W1)
    p = _sc_aggregate_128(h1, pk)
    h2 = _tc_layer2(p, h1, selfw2, b1.reshape(1, -1), W2)
    q = _sc_aggregate_64(h2, pk)
    return _tc_finish(q, h2, selfw2, b2.reshape(1, -1))
